# baseline (speedup 1.0000x reference)
import functools

import jax
import jax.numpy as jnp
from jax import lax
from jax.experimental import pallas as pl
from jax.experimental.pallas import tpu as pltpu
from jax.experimental.pallas import tpu_sc as plsc

_INFO = plsc.get_sparse_core_info()
_NC = _INFO.num_cores
_NW = _INFO.num_cores * _INFO.num_subcores

_V = 1000000
_D = 32
_B = 819200
_SEQ = 200
_BATCH = 4096

_CHUNK = 512
_UNITS = _B // _CHUNK
_UPW = _UNITS // _NW

_MESH = plsc.VectorSubcoreMesh(core_axis_name="c", subcore_axis_name="s")


@functools.partial(
    pl.kernel,
    mesh=_MESH,
    out_type=jax.ShapeDtypeStruct((_SEQ, 4, 32, 8, 128), jnp.float32),
    scratch_types=[
        [pltpu.VMEM((_CHUNK,), jnp.int32) for _ in range(2)],
        [pltpu.VMEM((_CHUNK, _D), jnp.float32) for _ in range(2)],
        [pltpu.VMEM((4, 4, 8, 128), jnp.float32) for _ in range(2)],
        [pltpu.SemaphoreType.DMA for _ in range(2)],
        [pltpu.SemaphoreType.DMA for _ in range(2)],
    ],
    compiler_params=pltpu.CompilerParams(
        use_tc_tiling_on_sc=False, needs_layout_passes=False
    ),
)
def _gather(xf_hbm, table_hbm, out5_hbm, idx, rows, fmt, gsem, wsem):
    w = lax.axis_index("s") * _NC + lax.axis_index("c")
    io16 = lax.iota(jnp.int32, 16)
    dvecs = [jnp.full((16,), d, jnp.int32) for d in range(_D)]

    def load_unit(u, b):
        pltpu.sync_copy(xf_hbm.at[pl.ds((u * _NW + w) * _CHUNK, _CHUNK)], idx[b])
        pltpu.async_copy(table_hbm.at[idx[b]], rows[b], gsem[b])

    def wait_gather(b):
        pltpu.make_async_copy(table_hbm.at[pl.ds(0, _CHUNK)], rows[b], gsem[b]).wait()

    def wait_writes(b):
        for tr in range(4):
            pltpu.make_async_copy(
                fmt[b].at[tr], out5_hbm.at[0, 0, pl.ds(0, 4)], wsem[b]
            ).wait()

    def transpose_unit(b):
        @plsc.parallel_loop(0, 1024, unroll=8)
        def _(i):
            d = i >> 5
            bvec = io16 + ((i & 31) << 4)
            dvec = jnp.full((16,), 0, jnp.int32) + d
            v = plsc.load_gather(rows[b], [bvec, dvec])
            fmt[b][d >> 3, (i >> 3) & 3, d & 7, pl.ds((i & 7) << 4, 16)] = v

    def write_unit(u, b):
        unit = u * _NW + w
        s = unit // 8
        tcg = unit % 8
        for tr in range(4):
            pltpu.async_copy(
                fmt[b].at[tr], out5_hbm.at[s, tr, pl.ds(tcg * 4, 4)], wsem[b]
            )

    load_unit(0, 0)

    def body(g, carry):
        for sb in range(2):
            u = g * 2 + sb

            @pl.when(u + 1 < _UPW)
            def _(nb=sb ^ 1, u=u):
                load_unit(u + 1, nb)

            wait_gather(sb)

            @pl.when(u >= 2)
            def _(sb=sb):
                wait_writes(sb)

            transpose_unit(sb)
            write_unit(u, sb)
        return carry

    lax.fori_loop(0, _UPW // 2, body, 0)
    wait_writes(0)
    wait_writes(1)


def kernel(x, table):
    t4 = lax.optimization_barrier(table.reshape(_V // 4, 128))
    t_lin = t4.reshape(_V, _D)
    xf = x.T.reshape(_B)
    out5 = _gather(xf, t_lin)
    return out5.transpose((2, 4, 0, 1, 3)).reshape(_BATCH, _SEQ, _D)


# speedup vs baseline: 1.0168x; 1.0168x over previous
import functools

import jax
import jax.numpy as jnp
from jax import lax
from jax.experimental import pallas as pl
from jax.experimental.pallas import tpu as pltpu
from jax.experimental.pallas import tpu_sc as plsc

_INFO = plsc.get_sparse_core_info()
_NC = _INFO.num_cores
_NW = _INFO.num_cores * _INFO.num_subcores

_V = 1000000
_D = 32
_B = 819200
_SEQ = 200
_BATCH = 4096

_CHUNK = 512
_UNITS = _B // _CHUNK
_UPW = _UNITS // _NW

_MESH = plsc.VectorSubcoreMesh(core_axis_name="c", subcore_axis_name="s")


@functools.partial(
    pl.kernel,
    mesh=_MESH,
    out_type=jax.ShapeDtypeStruct((_SEQ, 4, 32, 8, 128), jnp.float32),
    scratch_types=[
        [pltpu.VMEM((_CHUNK,), jnp.int32) for _ in range(2)],
        [pltpu.VMEM((_CHUNK, _D), jnp.float32) for _ in range(2)],
        [pltpu.VMEM((4, 4, 8, 128), jnp.float32) for _ in range(2)],
        [pltpu.SemaphoreType.DMA for _ in range(2)],
        [pltpu.SemaphoreType.DMA for _ in range(2)],
    ],
    compiler_params=pltpu.CompilerParams(
        use_tc_tiling_on_sc=False, needs_layout_passes=False
    ),
)
def _gather(xf_hbm, table_hbm, out5_hbm, idx, rows, fmt, gsem, wsem):
    w = lax.axis_index("s") * _NC + lax.axis_index("c")
    io16 = lax.iota(jnp.int32, 16)
    dvecs = [jnp.full((16,), d, jnp.int32) for d in range(_D)]

    def load_unit(u, b):
        pltpu.sync_copy(xf_hbm.at[pl.ds((u * _NW + w) * _CHUNK, _CHUNK)], idx[b])
        pltpu.async_copy(table_hbm.at[idx[b]], rows[b], gsem[b])

    def wait_gather(b):
        pltpu.make_async_copy(table_hbm.at[pl.ds(0, _CHUNK)], rows[b], gsem[b]).wait()

    def wait_writes(b):
        for tr in range(4):
            pltpu.make_async_copy(
                fmt[b].at[tr], out5_hbm.at[0, 0, pl.ds(0, 4)], wsem[b]
            ).wait()

    def transpose_unit(b):
        @plsc.parallel_loop(0, 32, unroll=4)
        def _(i):
            tcl = i >> 3
            k = i & 7
            bvec = io16 + ((tcl << 7) + (k << 4))
            for tr in range(4):
                vs = [
                    plsc.load_gather(rows[b], [bvec, dvecs[8 * tr + r]])
                    for r in range(8)
                ]
                for r in range(8):
                    fmt[b][tr, tcl, r, pl.ds(k << 4, 16)] = vs[r]

    def write_unit(u, b):
        unit = u * _NW + w
        s = unit // 8
        tcg = unit % 8
        for tr in range(4):
            pltpu.async_copy(
                fmt[b].at[tr], out5_hbm.at[s, tr, pl.ds(tcg * 4, 4)], wsem[b]
            )

    load_unit(0, 0)

    def body(g, carry):
        for sb in range(2):
            u = g * 2 + sb

            @pl.when(u + 1 < _UPW)
            def _(nb=sb ^ 1, u=u):
                load_unit(u + 1, nb)

            wait_gather(sb)

            @pl.when(u >= 2)
            def _(sb=sb):
                wait_writes(sb)

            transpose_unit(sb)
            write_unit(u, sb)
        return carry

    lax.fori_loop(0, _UPW // 2, body, 0)
    wait_writes(0)
    wait_writes(1)


def kernel(x, table):
    xf = x.T.reshape(_B)
    out5 = _gather(xf, table)
    return out5.transpose((2, 4, 0, 1, 3)).reshape(_BATCH, _SEQ, _D)


# speedup vs baseline: 1.1911x; 1.1715x over previous
import functools

import jax
import jax.numpy as jnp
from jax import lax
from jax.experimental import pallas as pl
from jax.experimental.pallas import tpu as pltpu
from jax.experimental.pallas import tpu_sc as plsc

_INFO = plsc.get_sparse_core_info()
_NC = _INFO.num_cores
_NW = _INFO.num_cores * _INFO.num_subcores

_V = 1000000
_D = 32
_B = 819200
_SEQ = 200
_BATCH = 4096

_CHUNK = 512
_UNITS = _B // _CHUNK
_UPW = _UNITS // _NW

_MESH = plsc.VectorSubcoreMesh(core_axis_name="c", subcore_axis_name="s")


@functools.partial(
    pl.kernel,
    mesh=_MESH,
    out_type=jax.ShapeDtypeStruct((_SEQ, 4, 32, 8, 128), jnp.float32),
    scratch_types=[
        pltpu.VMEM((_UPW * _CHUNK,), jnp.int32),
        [pltpu.VMEM((_CHUNK, _D), jnp.float32) for _ in range(2)],
        [pltpu.VMEM((4, 4, 8, 128), jnp.float32) for _ in range(2)],
        [pltpu.SemaphoreType.DMA for _ in range(2)],
        [pltpu.SemaphoreType.DMA for _ in range(2)],
    ],
    compiler_params=pltpu.CompilerParams(
        use_tc_tiling_on_sc=False, needs_layout_passes=False
    ),
)
def _gather(xf_hbm, table_hbm, out5_hbm, idx, rows, fmt, gsem, wsem):
    w = lax.axis_index("s") * _NC + lax.axis_index("c")
    io16 = lax.iota(jnp.int32, 16)
    dvecs = [jnp.full((16,), d, jnp.int32) for d in range(_D)]

    pltpu.sync_copy(xf_hbm.at[pl.ds(w * (_UPW * _CHUNK), _UPW * _CHUNK)], idx)

    def load_unit(u, b):
        pltpu.async_copy(
            table_hbm.at[idx.at[pl.ds(u * _CHUNK, _CHUNK)]], rows[b], gsem[b]
        )

    def wait_gather(b):
        pltpu.make_async_copy(table_hbm.at[pl.ds(0, _CHUNK)], rows[b], gsem[b]).wait()

    def wait_writes(b):
        for tr in range(4):
            pltpu.make_async_copy(
                fmt[b].at[tr], out5_hbm.at[0, 0, pl.ds(0, 4)], wsem[b]
            ).wait()

    def transpose_unit(b):
        @plsc.parallel_loop(0, 32, unroll=2)
        def _(i):
            tcl = i >> 3
            k = i & 7
            bvec = io16 + ((tcl << 7) + (k << 4))
            for tr in range(4):
                vs = [
                    plsc.load_gather(rows[b], [bvec, dvecs[8 * tr + r]])
                    for r in range(8)
                ]
                for r in range(8):
                    fmt[b][tr, tcl, r, pl.ds(k << 4, 16)] = vs[r]

    def write_unit(u, b):
        unit = w * _UPW + u
        s = unit // 8
        tcg = unit % 8
        for tr in range(4):
            pltpu.async_copy(
                fmt[b].at[tr], out5_hbm.at[s, tr, pl.ds(tcg * 4, 4)], wsem[b]
            )

    load_unit(0, 0)

    def body(g, carry):
        for sb in range(2):
            u = g * 2 + sb

            @pl.when(u + 1 < _UPW)
            def _(nb=sb ^ 1, u=u):
                load_unit(u + 1, nb)

            wait_gather(sb)

            @pl.when(u >= 2)
            def _(sb=sb):
                wait_writes(sb)

            transpose_unit(sb)
            write_unit(u, sb)
        return carry

    lax.fori_loop(0, _UPW // 2, body, 0)
    wait_writes(0)
    wait_writes(1)


def kernel(x, table):
    t4 = lax.optimization_barrier(table.reshape(_V // 4, 128))
    t_lin = t4.reshape(_V, _D)
    xf = x.T.reshape(_B)
    out5 = _gather(xf, t_lin)
    return out5.transpose((2, 4, 0, 1, 3)).reshape(_BATCH, _SEQ, _D)
